# baseline (speedup 1.0000x reference)
import functools

import jax
import jax.numpy as jnp
from jax import lax
from jax.experimental import pallas as pl
from jax.experimental.pallas import tpu as pltpu
from jax.experimental.pallas import tpu_sc as plsc

H = 128
SCALE = 30.0

_NC = 2
_NS = 16
_NW = _NC * _NS

_CHUNK = 128


def _gelu(x):
    return 0.5 * x * (1.0 + lax.erf(x * 0.7071067811865476))


def _ln(x, g, b, eps=1e-5):
    mu = jnp.mean(x, axis=-1, keepdims=True)
    var = jnp.mean((x - mu) ** 2, axis=-1, keepdims=True)
    return (x - mu) / jnp.sqrt(var + eps) * g + b


def _proj_body(hv_ref, w_ref, o_ref):
    o_ref[:] = jnp.dot(hv_ref[:], w_ref[:], preferred_element_type=jnp.float32)


def _project(h_V, W):
    L = h_V.shape[0]
    return pl.pallas_call(
        _proj_body,
        out_shape=jax.ShapeDtypeStruct((L, H), jnp.float32),
    )(h_V, W)


def _sc_gather(table, idx2d):
    R = idx2d.shape[0]
    assert R % (2 * _NW) == 0
    cpw = R // _NW
    half = cpw // 2

    mesh = plsc.VectorSubcoreMesh(core_axis_name="c", subcore_axis_name="s")

    @functools.partial(
        pl.kernel,
        mesh=mesh,
        out_type=jax.ShapeDtypeStruct((R * _CHUNK, H), jnp.float32),
        scratch_types=[
            pltpu.VMEM((cpw, _CHUNK), jnp.int32),
            pltpu.VMEM((2, _CHUNK, H), jnp.float32),
            pltpu.SemaphoreType.DMA,
            pltpu.SemaphoreType.DMA((2,)),
            pltpu.SemaphoreType.DMA((2,)),
        ],
    )
    def k(table_hbm, idx_hbm, out_hbm, idx_all, rows, isem, gsem, ssem):
        wid = lax.axis_index("s") * _NC + lax.axis_index("c")
        base = wid * cpw

        def gather(g, p):
            return pltpu.make_async_copy(
                table_hbm.at[idx_all.at[g]], rows.at[p], gsem.at[p])

        def store(g, p):
            return pltpu.make_async_copy(
                rows.at[p], out_hbm.at[pl.ds((base + g) * _CHUNK, _CHUNK)],
                ssem.at[p])

        idx_cp = pltpu.make_async_copy(idx_hbm.at[pl.ds(base, cpw)], idx_all,
                                       isem)
        idx_cp.start()
        idx_cp.wait()
        gather(0, 0).start()

        def body(t, carry):
            g0 = 2 * t
            g1 = g0 + 1
            gather(g0, 0).wait()
            store(g0, 0).start()

            @pl.when(t >= 1)
            def _():
                store(g1 - 2, 1).wait()

            gather(g1, 1).start()
            gather(g1, 1).wait()
            store(g1, 1).start()

            @pl.when(t < half - 1)
            def _():
                store(g0, 0).wait()
                gather(g0 + 2, 0).start()

            return carry

        lax.fori_loop(0, half, body, 0)
        store(cpw - 2, 0).wait()
        store(cpw - 1, 1).wait()

    return k(table, idx2d)


def _tc1_body(TL, K,
              hv_ref, he_ref, g1_ref,
              w1a, w1b, b1, w2, b2, w3, b3,
              w_in, b_in, w_out, b_out,
              g1g, bt1, g2g, bt2,
              w11a, w11c, b11,
              o_v, o_a2, o_c2):
    v = hv_ref[:]
    e = he_ref[:]
    g = g1_ref[:]
    a1 = jnp.dot(v, w1a[:], preferred_element_type=jnp.float32) + b1[:]
    pre = jnp.dot(e, w1b[:], preferred_element_type=jnp.float32) + g
    pre = (pre.reshape(TL, K, H) + a1[:, None, :]).reshape(TL * K, H)
    m = _gelu(pre)
    m = _gelu(jnp.dot(m, w2[:], preferred_element_type=jnp.float32) + b2[:])
    m = jnp.dot(m, w3[:], preferred_element_type=jnp.float32) + b3[:]
    dh = jnp.sum(m.reshape(TL, K, H), axis=1) * (1.0 / SCALE)
    x = _ln(v + dh, g1g[:], bt1[:])
    t = jnp.dot(_gelu(jnp.dot(x, w_in[:], preferred_element_type=jnp.float32) + b_in[:]),
                w_out[:], preferred_element_type=jnp.float32) + b_out[:]
    x = _ln(x + t, g2g[:], bt2[:])
    o_v[:] = x
    o_a2[:] = jnp.dot(x, w11a[:], preferred_element_type=jnp.float32) + b11[:]
    o_c2[:] = jnp.dot(x, w11c[:], preferred_element_type=jnp.float32)


def _tc2_body(TL, K,
              he_ref, g2_ref, a2_ref,
              w11b, w12, b12, w13, b13,
              g3g, bt3,
              o_e):
    e = he_ref[:]
    g = g2_ref[:]
    a2 = a2_ref[:]
    pre = jnp.dot(e, w11b[:], preferred_element_type=jnp.float32) + g
    pre = (pre.reshape(TL, K, H) + a2[:, None, :]).reshape(TL * K, H)
    m = _gelu(pre)
    m = _gelu(jnp.dot(m, w12[:], preferred_element_type=jnp.float32) + b12[:])
    m = jnp.dot(m, w13[:], preferred_element_type=jnp.float32) + b13[:]
    o_e[:] = _ln(e + m, g3g[:], bt3[:])


def _row(b):
    return b.reshape(1, -1)


def _pad_idx2d(idx_flat, n_table):
    n = idx_flat.shape[0]
    n_rows = -(-n // _CHUNK)
    R = ((n_rows + 8 * _NW - 1) // (8 * _NW)) * (8 * _NW)
    pad = R * _CHUNK - n
    if pad:
        idx_flat = jnp.concatenate(
            [idx_flat, (jnp.arange(pad, dtype=jnp.int32) * 127) % n_table])
    return idx_flat.reshape(R, _CHUNK)


def kernel(h_V, h_E, E_idx, W1, b1, W2, b2, W3, b3, W11, b11, W12, b12, W13,
           b13, W_in, b_in, W_out, b_out, g1, bt1, g2, bt2, g3, bt3):
    L, K = h_E.shape[0], h_E.shape[1]
    TL = 256
    n_parts = 4
    base = (L // (n_parts * TL)) * TL
    parts = [base] * (n_parts - 1) + [L - (n_parts - 1) * base]
    bounds = [0]
    for p in parts:
        bounds.append(bounds[-1] + p)

    W1a, W1b, W1c = W1[:H], W1[H:2 * H], W1[2 * H:]
    W11a, W11b, W11c = W11[:H], W11[H:2 * H], W11[2 * H:]

    he2 = h_E.reshape(L * K, H)
    idx_flat = E_idx.reshape(L * K)
    idx_parts = [_pad_idx2d(idx_flat[b0 * K:b1 * K], L)
                 for b0, b1 in zip(bounds[:-1], bounds[1:])]

    def wspec(a):
        return pl.BlockSpec(a.shape, lambda i: tuple(0 for _ in a.shape))

    def any_spec():
        return pl.BlockSpec(memory_space=pl.ANY)

    def nspec(off):
        return pl.BlockSpec((TL, H), lambda i, o=off: (i + o, 0))

    def espec(off):
        return pl.BlockSpec((TL * K, H), lambda i, o=off: (i + o, 0))

    weights1 = (W1a, W1b, _row(b1), W2, _row(b2), W3, _row(b3),
                W_in, _row(b_in), W_out, _row(b_out),
                _row(g1), _row(bt1), _row(g2), _row(bt2),
                W11a, W11c, _row(b11))
    weights2 = (W11b, W12, _row(b12), W13, _row(b13), _row(g3), _row(bt3))
    node_out = [jax.ShapeDtypeStruct((L, H), jnp.float32)] * 3
    edge_out = jax.ShapeDtypeStruct((L * K, H), jnp.float32)
    nw1 = len(weights1)

    def _tc1_alias_body(*refs):
        _tc1_body(TL, K, *refs[:3 + nw1], *refs[6 + nw1:])

    def _tc2_alias_body(*refs):
        _tc2_body(TL, K, *refs[:3 + len(weights2)], *refs[4 + len(weights2):])

    c1 = _project(h_V, W1c)
    g1s = [_sc_gather(c1, ip) for ip in idx_parts]

    prev = None
    for p in range(n_parts):
        off = bounds[p] // TL
        nt = -(-parts[p] // TL)
        specs = [nspec(off), espec(off), espec(0)] + [wspec(w) for w in weights1]
        if prev is None:
            prev = pl.pallas_call(
                functools.partial(_tc1_body, TL, K),
                grid=(nt,), in_specs=specs,
                out_specs=[nspec(off)] * 3, out_shape=node_out,
            )(h_V, he2, g1s[p], *weights1)
        else:
            prev = pl.pallas_call(
                _tc1_alias_body,
                grid=(nt,), in_specs=specs + [any_spec()] * 3,
                out_specs=[nspec(off)] * 3, out_shape=node_out,
                input_output_aliases={3 + nw1: 0, 4 + nw1: 1, 5 + nw1: 2},
            )(h_V, he2, g1s[p], *weights1, *prev)
    h_V2, a2, c2 = prev

    g2s = [_sc_gather(c2, ip) for ip in idx_parts]

    prev_e = None
    for p in range(n_parts):
        off = bounds[p] // TL
        nt = -(-parts[p] // TL)
        specs = [espec(off), espec(0), nspec(off)] + [wspec(w) for w in weights2]
        if prev_e is None:
            prev_e = pl.pallas_call(
                functools.partial(_tc2_body, TL, K),
                grid=(nt,), in_specs=specs,
                out_specs=espec(off), out_shape=edge_out,
            )(he2, g2s[p], a2, *weights2)
        else:
            prev_e = pl.pallas_call(
                _tc2_alias_body,
                grid=(nt,), in_specs=specs + [any_spec()],
                out_specs=espec(off), out_shape=edge_out,
                input_output_aliases={3 + len(weights2): 0},
            )(he2, g2s[p], a2, *weights2, prev_e)

    return h_V2, prev_e.reshape(L, K, H)


# speedup vs baseline: 1.0819x; 1.0819x over previous
import functools

import jax
import jax.numpy as jnp
from jax import lax
from jax.experimental import pallas as pl
from jax.experimental.pallas import tpu as pltpu
from jax.experimental.pallas import tpu_sc as plsc

H = 128
SCALE = 30.0

_NC = 2
_NS = 16
_NW = _NC * _NS

_CHUNK = 128


def _gelu(x):
    return 0.5 * x * (1.0 + lax.erf(x * 0.7071067811865476))


def _ln(x, g, b, eps=1e-5):
    mu = jnp.mean(x, axis=-1, keepdims=True)
    var = jnp.mean((x - mu) ** 2, axis=-1, keepdims=True)
    return (x - mu) / jnp.sqrt(var + eps) * g + b


def _proj_body(hv_ref, w_ref, o_ref):
    o_ref[:] = jnp.dot(hv_ref[:], w_ref[:], preferred_element_type=jnp.float32)


def _project(h_V, W):
    L = h_V.shape[0]
    return pl.pallas_call(
        _proj_body,
        out_shape=jax.ShapeDtypeStruct((L, H), jnp.float32),
    )(h_V, W)


def _sc_gather(table, idx2d):
    R = idx2d.shape[0]
    assert R % (2 * _NW) == 0
    cpw = R // _NW
    half = cpw // 2

    mesh = plsc.VectorSubcoreMesh(core_axis_name="c", subcore_axis_name="s")

    @functools.partial(
        pl.kernel,
        mesh=mesh,
        out_type=jax.ShapeDtypeStruct((R * _CHUNK, H), jnp.float32),
        scratch_types=[
            pltpu.VMEM((cpw, _CHUNK), jnp.int32),
            pltpu.VMEM((2, _CHUNK, H), jnp.float32),
            pltpu.SemaphoreType.DMA,
            pltpu.SemaphoreType.DMA((2,)),
            pltpu.SemaphoreType.DMA((2,)),
        ],
    )
    def k(table_hbm, idx_hbm, out_hbm, idx_all, rows, isem, gsem, ssem):
        wid = lax.axis_index("s") * _NC + lax.axis_index("c")
        base = wid * cpw

        def gather(g, p):
            return pltpu.make_async_copy(
                table_hbm.at[idx_all.at[g]], rows.at[p], gsem.at[p])

        def store(g, p):
            return pltpu.make_async_copy(
                rows.at[p], out_hbm.at[pl.ds((base + g) * _CHUNK, _CHUNK)],
                ssem.at[p])

        idx_cp = pltpu.make_async_copy(idx_hbm.at[pl.ds(base, cpw)], idx_all,
                                       isem)
        idx_cp.start()
        idx_cp.wait()
        gather(0, 0).start()

        def body(t, carry):
            g0 = 2 * t
            g1 = g0 + 1
            gather(g0, 0).wait()
            store(g0, 0).start()

            @pl.when(t >= 1)
            def _():
                store(g1 - 2, 1).wait()

            gather(g1, 1).start()
            gather(g1, 1).wait()
            store(g1, 1).start()

            @pl.when(t < half - 1)
            def _():
                store(g0, 0).wait()
                gather(g0 + 2, 0).start()

            return carry

        lax.fori_loop(0, half, body, 0)
        store(cpw - 2, 0).wait()
        store(cpw - 1, 1).wait()

    return k(table, idx2d)


def _tc1_body(TL, K,
              hv_ref, he_ref, g1_ref,
              w1a, w1b, b1, w2, b2, w3, b3,
              w_in, b_in, w_out, b_out,
              g1g, bt1, g2g, bt2,
              w11a, w11c, b11,
              o_v, o_a2, o_c2):
    v = hv_ref[:]
    e = he_ref[:]
    g = g1_ref[:]
    a1 = jnp.dot(v, w1a[:], preferred_element_type=jnp.float32) + b1[:]
    pre = jnp.dot(e, w1b[:], preferred_element_type=jnp.float32) + g
    pre = (pre.reshape(TL, K, H) + a1[:, None, :]).reshape(TL * K, H)
    m = _gelu(pre)
    m = _gelu(jnp.dot(m, w2[:], preferred_element_type=jnp.float32) + b2[:])
    m = jnp.dot(m, w3[:], preferred_element_type=jnp.float32) + b3[:]
    dh = jnp.sum(m.reshape(TL, K, H), axis=1) * (1.0 / SCALE)
    x = _ln(v + dh, g1g[:], bt1[:])
    t = jnp.dot(_gelu(jnp.dot(x, w_in[:], preferred_element_type=jnp.float32) + b_in[:]),
                w_out[:], preferred_element_type=jnp.float32) + b_out[:]
    x = _ln(x + t, g2g[:], bt2[:])
    o_v[:] = x
    o_a2[:] = jnp.dot(x, w11a[:], preferred_element_type=jnp.float32) + b11[:]
    o_c2[:] = jnp.dot(x, w11c[:], preferred_element_type=jnp.float32)


def _tc2_body(TL, K,
              he_ref, g2_ref, a2_ref,
              w11b, w12, b12, w13, b13,
              g3g, bt3,
              o_e):
    e = he_ref[:]
    g = g2_ref[:]
    a2 = a2_ref[:]
    pre = jnp.dot(e, w11b[:], preferred_element_type=jnp.float32) + g
    pre = (pre.reshape(TL, K, H) + a2[:, None, :]).reshape(TL * K, H)
    m = _gelu(pre)
    m = _gelu(jnp.dot(m, w12[:], preferred_element_type=jnp.float32) + b12[:])
    m = jnp.dot(m, w13[:], preferred_element_type=jnp.float32) + b13[:]
    o_e[:] = _ln(e + m, g3g[:], bt3[:])


def _row(b):
    return b.reshape(1, -1)


def _pad_idx2d(idx_flat, n_table):
    n = idx_flat.shape[0]
    n_rows = -(-n // _CHUNK)
    R = ((n_rows + 8 * _NW - 1) // (8 * _NW)) * (8 * _NW)
    pad = R * _CHUNK - n
    if pad:
        idx_flat = jnp.concatenate(
            [idx_flat, (jnp.arange(pad, dtype=jnp.int32) * 127) % n_table])
    return idx_flat.reshape(R, _CHUNK)


def kernel(h_V, h_E, E_idx, W1, b1, W2, b2, W3, b3, W11, b11, W12, b12, W13,
           b13, W_in, b_in, W_out, b_out, g1, bt1, g2, bt2, g3, bt3):
    L, K = h_E.shape[0], h_E.shape[1]
    TL = 200
    n_parts = 2
    base = (L // (n_parts * TL)) * TL
    parts = [base] * (n_parts - 1) + [L - (n_parts - 1) * base]
    bounds = [0]
    for p in parts:
        bounds.append(bounds[-1] + p)

    W1a, W1b, W1c = W1[:H], W1[H:2 * H], W1[2 * H:]
    W11a, W11b, W11c = W11[:H], W11[H:2 * H], W11[2 * H:]

    he2 = h_E.reshape(L * K, H)
    idx_flat = E_idx.reshape(L * K)
    idx_parts = [_pad_idx2d(idx_flat[b0 * K:b1 * K], L)
                 for b0, b1 in zip(bounds[:-1], bounds[1:])]

    def wspec(a):
        return pl.BlockSpec(a.shape, lambda i: tuple(0 for _ in a.shape))

    def any_spec():
        return pl.BlockSpec(memory_space=pl.ANY)

    def nspec(off):
        return pl.BlockSpec((TL, H), lambda i, o=off: (i + o, 0))

    def espec(off):
        return pl.BlockSpec((TL * K, H), lambda i, o=off: (i + o, 0))

    weights1 = (W1a, W1b, _row(b1), W2, _row(b2), W3, _row(b3),
                W_in, _row(b_in), W_out, _row(b_out),
                _row(g1), _row(bt1), _row(g2), _row(bt2),
                W11a, W11c, _row(b11))
    weights2 = (W11b, W12, _row(b12), W13, _row(b13), _row(g3), _row(bt3))
    node_out = [jax.ShapeDtypeStruct((L, H), jnp.float32)] * 3
    edge_out = jax.ShapeDtypeStruct((L * K, H), jnp.float32)
    nw1 = len(weights1)

    def _tc1_alias_body(*refs):
        _tc1_body(TL, K, *refs[:3 + nw1], *refs[6 + nw1:])

    def _tc2_alias_body(*refs):
        _tc2_body(TL, K, *refs[:3 + len(weights2)], *refs[4 + len(weights2):])

    c1 = _project(h_V, W1c)
    g1s = [_sc_gather(c1, ip) for ip in idx_parts]

    prev = None
    for p in range(n_parts):
        off = bounds[p] // TL
        nt = -(-parts[p] // TL)
        specs = [nspec(off), espec(off), espec(0)] + [wspec(w) for w in weights1]
        if prev is None:
            prev = pl.pallas_call(
                functools.partial(_tc1_body, TL, K),
                grid=(nt,), in_specs=specs,
                out_specs=[nspec(off)] * 3, out_shape=node_out,
            )(h_V, he2, g1s[p], *weights1)
        else:
            prev = pl.pallas_call(
                _tc1_alias_body,
                grid=(nt,), in_specs=specs + [any_spec()] * 3,
                out_specs=[nspec(off)] * 3, out_shape=node_out,
                input_output_aliases={3 + nw1: 0, 4 + nw1: 1, 5 + nw1: 2},
            )(h_V, he2, g1s[p], *weights1, *prev)
    h_V2, a2, c2 = prev

    g2s = [_sc_gather(c2, ip) for ip in idx_parts]

    prev_e = None
    for p in range(n_parts):
        off = bounds[p] // TL
        nt = -(-parts[p] // TL)
        specs = [espec(off), espec(0), nspec(off)] + [wspec(w) for w in weights2]
        if prev_e is None:
            prev_e = pl.pallas_call(
                functools.partial(_tc2_body, TL, K),
                grid=(nt,), in_specs=specs,
                out_specs=espec(off), out_shape=edge_out,
            )(he2, g2s[p], a2, *weights2)
        else:
            prev_e = pl.pallas_call(
                _tc2_alias_body,
                grid=(nt,), in_specs=specs + [any_spec()],
                out_specs=espec(off), out_shape=edge_out,
                input_output_aliases={3 + len(weights2): 0},
            )(he2, g2s[p], a2, *weights2, prev_e)

    return h_V2, prev_e.reshape(L, K, H)


# speedup vs baseline: 1.0973x; 1.0142x over previous
import functools

import jax
import jax.numpy as jnp
from jax import lax
from jax.experimental import pallas as pl
from jax.experimental.pallas import tpu as pltpu
from jax.experimental.pallas import tpu_sc as plsc

H = 128
SCALE = 30.0

_NC = 2
_NS = 16
_NW = _NC * _NS

_CHUNK = 128


def _gelu(x):
    return 0.5 * x * (1.0 + lax.erf(x * 0.7071067811865476))


def _ln(x, g, b, eps=1e-5):
    mu = jnp.mean(x, axis=-1, keepdims=True)
    var = jnp.mean((x - mu) ** 2, axis=-1, keepdims=True)
    return (x - mu) / jnp.sqrt(var + eps) * g + b


def _proj_body(hv_ref, w_ref, o_ref):
    o_ref[:] = jnp.dot(hv_ref[:], w_ref[:], preferred_element_type=jnp.float32)


def _project(h_V, W):
    L = h_V.shape[0]
    return pl.pallas_call(
        _proj_body,
        out_shape=jax.ShapeDtypeStruct((L, H), jnp.float32),
    )(h_V, W)


def _sc_gather(table, idx2d):
    R = idx2d.shape[0]
    assert R % (2 * _NW) == 0
    cpw = R // _NW
    half = cpw // 2

    mesh = plsc.VectorSubcoreMesh(core_axis_name="c", subcore_axis_name="s")

    @functools.partial(
        pl.kernel,
        mesh=mesh,
        out_type=jax.ShapeDtypeStruct((R * _CHUNK, H), jnp.float32),
        scratch_types=[
            pltpu.VMEM((cpw, _CHUNK), jnp.int32),
            pltpu.VMEM((2, _CHUNK, H), jnp.float32),
            pltpu.SemaphoreType.DMA,
            pltpu.SemaphoreType.DMA((2,)),
            pltpu.SemaphoreType.DMA((2,)),
        ],
    )
    def k(table_hbm, idx_hbm, out_hbm, idx_all, rows, isem, gsem, ssem):
        wid = lax.axis_index("s") * _NC + lax.axis_index("c")
        base = wid * cpw

        def gather(g, p):
            return pltpu.make_async_copy(
                table_hbm.at[idx_all.at[g]], rows.at[p], gsem.at[p])

        def store(g, p):
            return pltpu.make_async_copy(
                rows.at[p], out_hbm.at[pl.ds((base + g) * _CHUNK, _CHUNK)],
                ssem.at[p])

        idx_cp = pltpu.make_async_copy(idx_hbm.at[pl.ds(base, cpw)], idx_all,
                                       isem)
        idx_cp.start()
        idx_cp.wait()
        gather(0, 0).start()

        def body(t, carry):
            g0 = 2 * t
            g1 = g0 + 1
            gather(g0, 0).wait()
            store(g0, 0).start()

            @pl.when(t >= 1)
            def _():
                store(g1 - 2, 1).wait()

            gather(g1, 1).start()
            gather(g1, 1).wait()
            store(g1, 1).start()

            @pl.when(t < half - 1)
            def _():
                store(g0, 0).wait()
                gather(g0 + 2, 0).start()

            return carry

        lax.fori_loop(0, half, body, 0)
        store(cpw - 2, 0).wait()
        store(cpw - 1, 1).wait()

    return k(table, idx2d)


def _tc1_body(TL, K,
              hv_ref, he_ref, g1_ref,
              w1a, w1b, b1, w2, b2, w3, b3,
              w_in, b_in, w_out, b_out,
              g1g, bt1, g2g, bt2,
              w11a, w11c, b11,
              o_v, o_a2, o_c2):
    v = hv_ref[:]
    e = he_ref[:]
    g = g1_ref[:]
    a1 = jnp.dot(v, w1a[:], preferred_element_type=jnp.float32) + b1[:]
    pre = jnp.dot(e, w1b[:], preferred_element_type=jnp.float32) + g
    pre = (pre.reshape(TL, K, H) + a1[:, None, :]).reshape(TL * K, H)
    m = _gelu(pre)
    m = _gelu(jnp.dot(m, w2[:], preferred_element_type=jnp.float32) + b2[:])
    m = jnp.dot(m, w3[:], preferred_element_type=jnp.float32) + b3[:]
    dh = jnp.sum(m.reshape(TL, K, H), axis=1) * (1.0 / SCALE)
    x = _ln(v + dh, g1g[:], bt1[:])
    t = jnp.dot(_gelu(jnp.dot(x, w_in[:], preferred_element_type=jnp.float32) + b_in[:]),
                w_out[:], preferred_element_type=jnp.float32) + b_out[:]
    x = _ln(x + t, g2g[:], bt2[:])
    o_v[:] = x
    o_a2[:] = jnp.dot(x, w11a[:], preferred_element_type=jnp.float32) + b11[:]
    o_c2[:] = jnp.dot(x, w11c[:], preferred_element_type=jnp.float32)


def _tc2_body(TL, K,
              he_ref, g2_ref, a2_ref,
              w11b, w12, b12, w13, b13,
              g3g, bt3,
              o_e):
    e = he_ref[:]
    g = g2_ref[:]
    a2 = a2_ref[:]
    pre = jnp.dot(e, w11b[:], preferred_element_type=jnp.float32) + g
    pre = (pre.reshape(TL, K, H) + a2[:, None, :]).reshape(TL * K, H)
    m = _gelu(pre)
    m = _gelu(jnp.dot(m, w12[:], preferred_element_type=jnp.float32) + b12[:])
    m = jnp.dot(m, w13[:], preferred_element_type=jnp.float32) + b13[:]
    o_e[:] = _ln(e + m, g3g[:], bt3[:])


def _row(b):
    return b.reshape(1, -1)


def _pad_idx2d(idx_flat, n_table):
    n = idx_flat.shape[0]
    n_rows = -(-n // _CHUNK)
    R = ((n_rows + 8 * _NW - 1) // (8 * _NW)) * (8 * _NW)
    pad = R * _CHUNK - n
    if pad:
        idx_flat = jnp.concatenate(
            [idx_flat, (jnp.arange(pad, dtype=jnp.int32) * 127) % n_table])
    return idx_flat.reshape(R, _CHUNK)


def kernel(h_V, h_E, E_idx, W1, b1, W2, b2, W3, b3, W11, b11, W12, b12, W13,
           b13, W_in, b_in, W_out, b_out, g1, bt1, g2, bt2, g3, bt3):
    L, K = h_E.shape[0], h_E.shape[1]
    TL = 400
    n_parts = 2
    base = (L // (n_parts * TL)) * TL
    parts = [base] * (n_parts - 1) + [L - (n_parts - 1) * base]
    bounds = [0]
    for p in parts:
        bounds.append(bounds[-1] + p)

    W1a, W1b, W1c = W1[:H], W1[H:2 * H], W1[2 * H:]
    W11a, W11b, W11c = W11[:H], W11[H:2 * H], W11[2 * H:]

    he2 = h_E.reshape(L * K, H)
    idx_flat = E_idx.reshape(L * K)
    idx_parts = [_pad_idx2d(idx_flat[b0 * K:b1 * K], L)
                 for b0, b1 in zip(bounds[:-1], bounds[1:])]

    def wspec(a):
        return pl.BlockSpec(a.shape, lambda i: tuple(0 for _ in a.shape))

    def any_spec():
        return pl.BlockSpec(memory_space=pl.ANY)

    def nspec(off):
        return pl.BlockSpec((TL, H), lambda i, o=off: (i + o, 0))

    def espec(off):
        return pl.BlockSpec((TL * K, H), lambda i, o=off: (i + o, 0))

    weights1 = (W1a, W1b, _row(b1), W2, _row(b2), W3, _row(b3),
                W_in, _row(b_in), W_out, _row(b_out),
                _row(g1), _row(bt1), _row(g2), _row(bt2),
                W11a, W11c, _row(b11))
    weights2 = (W11b, W12, _row(b12), W13, _row(b13), _row(g3), _row(bt3))
    node_out = [jax.ShapeDtypeStruct((L, H), jnp.float32)] * 3
    edge_out = jax.ShapeDtypeStruct((L * K, H), jnp.float32)
    nw1 = len(weights1)

    def _tc1_alias_body(*refs):
        _tc1_body(TL, K, *refs[:3 + nw1], *refs[6 + nw1:])

    def _tc2_alias_body(*refs):
        _tc2_body(TL, K, *refs[:3 + len(weights2)], *refs[4 + len(weights2):])

    c1 = _project(h_V, W1c)
    g1s = [_sc_gather(c1, ip) for ip in idx_parts]

    prev = None
    for p in range(n_parts):
        off = bounds[p] // TL
        nt = -(-parts[p] // TL)
        specs = [nspec(off), espec(off), espec(0)] + [wspec(w) for w in weights1]
        if prev is None:
            prev = pl.pallas_call(
                functools.partial(_tc1_body, TL, K),
                grid=(nt,), in_specs=specs,
                out_specs=[nspec(off)] * 3, out_shape=node_out,
            )(h_V, he2, g1s[p], *weights1)
        else:
            prev = pl.pallas_call(
                _tc1_alias_body,
                grid=(nt,), in_specs=specs + [any_spec()] * 3,
                out_specs=[nspec(off)] * 3, out_shape=node_out,
                input_output_aliases={3 + nw1: 0, 4 + nw1: 1, 5 + nw1: 2},
            )(h_V, he2, g1s[p], *weights1, *prev)
    h_V2, a2, c2 = prev

    g2s = [_sc_gather(c2, ip) for ip in idx_parts]

    prev_e = None
    for p in range(n_parts):
        off = bounds[p] // TL
        nt = -(-parts[p] // TL)
        specs = [espec(off), espec(0), nspec(off)] + [wspec(w) for w in weights2]
        if prev_e is None:
            prev_e = pl.pallas_call(
                functools.partial(_tc2_body, TL, K),
                grid=(nt,), in_specs=specs,
                out_specs=espec(off), out_shape=edge_out,
            )(he2, g2s[p], a2, *weights2)
        else:
            prev_e = pl.pallas_call(
                _tc2_alias_body,
                grid=(nt,), in_specs=specs + [any_spec()],
                out_specs=espec(off), out_shape=edge_out,
                input_output_aliases={3 + len(weights2): 0},
            )(he2, g2s[p], a2, *weights2, prev_e)

    return h_V2, prev_e.reshape(L, K, H)
